# Initial kernel scaffold; baseline (speedup 1.0000x reference)
import functools
import math

import jax
import jax.numpy as jnp
from jax import lax
from jax.experimental import pallas as pl
from jax.experimental.pallas import tpu as pltpu
from jax.experimental.pallas import tpu_sc as plsc

N = 10000
ERAW = 160000
ESL = ERAW + N
EP = 172032
NC = 2
NS = 16
B = 128
EPT = EP // (NC * NS)
NB = EPT // B
EPT2 = EP // NS
NB2 = EPT2 // B
RPT = N // NS
LOG2PI = math.log(2.0 * math.pi)
F32 = jnp.float32
I32 = jnp.int32

_mesh = functools.partial(
    plsc.VectorSubcoreMesh, core_axis_name="c", subcore_axis_name="s")


def _zero16(ref, nrows):
  def zr(i, _):
    ref[i, :] = jnp.zeros((16,), F32)
    return 0
  lax.fori_loop(0, nrows, zr, 0)


def _zero_rows128(ref, nrows):
  def zr(i, _):
    for f in range(8):
      ref[i, pl.ds(f * 16, 16)] = jnp.zeros((16,), F32)
    return 0
  lax.fori_loop(0, nrows, zr, 0)


def _b1_call(H, td, ts, src, dst):
  def body(td_hbm, ts_hbm, src_hbm, dst_hbm, statsp_hbm, ex_hbm,
           td_v, ts_v, srcb, dstb, contrib, exbuf, zbuf, acc_sh):
    cid = lax.axis_index("c")
    sid = lax.axis_index("s")
    wid = cid * NS + sid
    pltpu.sync_copy(td_hbm, td_v)
    pltpu.sync_copy(ts_hbm, ts_v)
    _zero16(contrib, B)
    _zero16(zbuf, RPT)
    pltpu.sync_copy(zbuf, acc_sh.at[pl.ds(sid * RPT, RPT)])
    plsc.subcore_barrier()

    def batch(b, _):
      base = wid * EPT + b * B
      pltpu.sync_copy(src_hbm.at[pl.ds(base, B)], srcb)
      pltpu.sync_copy(dst_hbm.at[pl.ds(base, B)], dstb)
      for g in range(B // 16):
        sv = srcb[pl.ds(g * 16, 16)]
        dv = dstb[pl.ds(g * 16, 16)]
        ev = base + g * 16 + lax.iota(I32, 16)
        validf = jnp.where(ev < ERAW, sv != dv, ev < ESL).astype(F32)
        rows = g * 16 + lax.iota(I32, 16)
        for h in range(H):
          colh = jnp.full((16,), h, I32)
          ai = plsc.load_gather(td_v, [dv * H + colh])
          aj = plsc.load_gather(ts_v, [sv * H + colh])
          s = ai + aj
          ar = jnp.where(s >= 0, s, 0.2 * s)
          ex = jnp.exp(ar) * validf
          plsc.store_scatter(contrib, [rows, colh], ex)
          plsc.store_scatter(contrib, [rows, jnp.full((16,), 4 + h, I32)],
                             ex * ar)
          exbuf[h, pl.ds(g * 16, 16)] = ex
        plsc.store_scatter(contrib, [rows, jnp.full((16,), 8, I32)], validf)
      for h in range(H):
        pltpu.sync_copy(exbuf.at[h], ex_hbm.at[h, pl.ds(base, B)])
      pltpu.sync_copy(contrib, acc_sh.at[dstb], add=True)
      return 0

    lax.fori_loop(0, NB, batch, 0)
    plsc.subcore_barrier()
    pltpu.sync_copy(acc_sh.at[pl.ds(sid * RPT, RPT)], zbuf)
    pltpu.sync_copy(zbuf, statsp_hbm.at[cid, pl.ds(sid * RPT, RPT)])

  fn = pl.kernel(
      body,
      out_type=[jax.ShapeDtypeStruct((NC, N, 16), F32),
                jax.ShapeDtypeStruct((H, EP), F32)],
      mesh=_mesh(),
      scratch_types=[
          pltpu.VMEM((N * H,), F32), pltpu.VMEM((N * H,), F32),
          pltpu.VMEM((B,), I32), pltpu.VMEM((B,), I32),
          pltpu.VMEM((B, 16), F32), pltpu.VMEM((H, B), F32),
          pltpu.VMEM((RPT, 16), F32),
          pltpu.MemorySpace.VMEM_SHARED((N, 16), F32),
      ],
  )
  return fn(td, ts, src, dst)


def _tm_call(H, statsp):
  def body(p_ref, m_ref, d_ref):
    m = p_ref[0] + p_ref[1]
    m_ref[...] = m
    d_ref[...] = m[:, 0:H]

  return pl.pallas_call(
      body,
      grid=(10,),
      in_specs=[pl.BlockSpec((NC, 1000, 16), lambda i: (0, i, 0))],
      out_specs=[pl.BlockSpec((1000, 16), lambda i: (i, 0)),
                 pl.BlockSpec((1000, H), lambda i: (i, 0))],
      out_shape=[jax.ShapeDtypeStruct((N, 16), F32),
                 jax.ShapeDtypeStruct((N, H), F32)],
  )(statsp)


def _b2_call(H, den, dst, ex):
  def body(den_hbm, dst_hbm, ex_hbm, alpha_hbm, den_v, dstb, exb, alb):
    cid = lax.axis_index("c")
    sid = lax.axis_index("s")
    wid = cid * NS + sid
    pltpu.sync_copy(den_hbm, den_v)

    def batch(b, _):
      base = wid * EPT + b * B
      pltpu.sync_copy(dst_hbm.at[pl.ds(base, B)], dstb)
      for h in range(H):
        pltpu.sync_copy(ex_hbm.at[h, pl.ds(base, B)], exb.at[h])
      for g in range(B // 16):
        dv = dstb[pl.ds(g * 16, 16)]
        for h in range(H):
          colh = jnp.full((16,), h, I32)
          dn = plsc.load_gather(den_v, [dv * H + colh])
          exv = exb[h, pl.ds(g * 16, 16)]
          alb[h, pl.ds(g * 16, 16)] = exv / (dn + 1e-16)
      for h in range(H):
        pltpu.sync_copy(alb.at[h], alpha_hbm.at[h, pl.ds(base, B)])
      return 0

    lax.fori_loop(0, NB, batch, 0)

  fn = pl.kernel(
      body,
      out_type=[jax.ShapeDtypeStruct((H, EP), F32)],
      mesh=_mesh(),
      scratch_types=[
          pltpu.VMEM((N * H,), F32),
          pltpu.VMEM((B,), I32),
          pltpu.VMEM((H, B), F32), pltpu.VMEM((H, B), F32),
      ],
  )
  return fn(den, dst, ex)[0]


def _c_call(nchunk, H, xw, src, dst, alpha):
  cpsc = nchunk // NC
  cph = nchunk // H

  def body(xw_hbm, src_hbm, dst_hbm, alpha_hbm, out_hbm,
           srcb, src2, dstb, alb, rows_v, zbuf, stg, acc_sh):
    cid = lax.axis_index("c")
    sid = lax.axis_index("s")
    _zero_rows128(zbuf, 125)
    for cc in range(cpsc):
      for z in range(5):
        pltpu.sync_copy(zbuf, acc_sh.at[pl.ds(sid * RPT + z * 125, 125)])
      plsc.subcore_barrier()
      chunk = cid * cpsc + cc

      def batch(b, _):
        base = sid * EPT2 + b * B
        pltpu.sync_copy(src_hbm.at[pl.ds(base, B)], srcb)
        pltpu.sync_copy(dst_hbm.at[pl.ds(base, B)], dstb)
        hh = (cid * cpsc + cc) // cph
        pltpu.sync_copy(alpha_hbm.at[hh, pl.ds(base, B)], alb)
        coff = (cid * cpsc + cc) * N
        for g in range(B // 16):
          src2[pl.ds(g * 16, 16)] = srcb[pl.ds(g * 16, 16)] + coff
        pltpu.sync_copy(xw_hbm.at[src2], rows_v)

        def scale(e, _):
          av = alb[e]
          for f in range(8):
            rows_v[e, pl.ds(f * 16, 16)] = rows_v[e, pl.ds(f * 16, 16)] * av
          return 0

        lax.fori_loop(0, B, scale, 0)
        pltpu.sync_copy(rows_v, acc_sh.at[dstb], add=True)
        return 0

      lax.fori_loop(0, NB2, batch, 0)
      plsc.subcore_barrier()
      for z in range(5):
        pltpu.sync_copy(acc_sh.at[pl.ds(sid * RPT + z * 125, 125)], stg)
        pltpu.sync_copy(
            stg, out_hbm.at[pl.ds(chunk * N + sid * RPT + z * 125, 125)])
      plsc.subcore_barrier()

  fn = pl.kernel(
      body,
      out_type=[jax.ShapeDtypeStruct((nchunk * N, 128), F32)],
      mesh=_mesh(),
      scratch_types=[
          pltpu.VMEM((B,), I32), pltpu.VMEM((B,), I32),
          pltpu.VMEM((B,), I32), pltpu.VMEM((B,), F32),
          pltpu.VMEM((B, 128), F32),
          pltpu.VMEM((125, 128), F32), pltpu.VMEM((125, 128), F32),
          pltpu.MemorySpace.VMEM_SHARED((N, 128), F32),
      ],
  )
  return fn(xw, src, dst, alpha)[0]


def _t1_call(x, W1, ap1):
  def body(x_ref, w_ref, ap_ref, xw_ref, ai_ref, aj_ref):
    c = pl.program_id(1)
    xc = jnp.dot(x_ref[...], w_ref[...], preferred_element_type=F32)
    xw_ref[...] = xc
    part = jnp.dot(xc, ap_ref[...], preferred_element_type=F32)

    @pl.when(c == 0)
    def _():
      ai_ref[...] = part[:, 0:4]
      aj_ref[...] = part[:, 4:8]

    @pl.when(c != 0)
    def _():
      ai_ref[...] += part[:, 0:4]
      aj_ref[...] += part[:, 4:8]

  return pl.pallas_call(
      body,
      grid=(10, 8),
      in_specs=[
          pl.BlockSpec((1000, 128), lambda i, c: (i, 0)),
          pl.BlockSpec((128, 128), lambda i, c: (0, c)),
          pl.BlockSpec((128, 8), lambda i, c: (c, 0)),
      ],
      out_specs=[
          pl.BlockSpec((1000, 128), lambda i, c: (c * 10 + i, 0)),
          pl.BlockSpec((1000, 4), lambda i, c: (i, 0)),
          pl.BlockSpec((1000, 4), lambda i, c: (i, 0)),
      ],
      out_shape=[jax.ShapeDtypeStruct((8 * N, 128), F32),
                 jax.ShapeDtypeStruct((N, 4), F32),
                 jax.ShapeDtypeStruct((N, 4), F32)],
  )(x, W1, ap1)


def _node_stage(nchunk, H, is_t2, out_flat, eps2d, pm, pls, logw,
                stats, w_next, ap_next, fcb):
  bn = 400
  nblocks = N // bn
  d_model = nchunk * 128

  def body(*refs):
    outr = refs[0:nchunk]
    epsr = refs[nchunk:2 * nchunk]
    pm_ref, pls_ref, lw_ref, st_ref, wn_ref = refs[2 * nchunk:2 * nchunk + 5]
    k = 2 * nchunk + 5
    if is_t2:
      ap_ref = refs[k]
      xw2_ref, ai2_ref, aj2_ref, kl_ref, nr_ref, lq_ref, lp_ref = refs[k + 1:]
    else:
      fcb_ref = refs[k]
      logit_ref, kl_ref, lq_ref, lp_ref = refs[k + 1:]
      nr_ref = None
    i = pl.program_id(0)

    st = st_ref[...]
    den = st[:, 0:H]
    s1 = st[:, 4:4 + H]
    deg = st[:, 8:9]
    klb = jnp.sum(s1 / den - jnp.log(den)) + H * jnp.sum(jnp.log(deg))

    lqb = jnp.float32(-0.5 * d_model * LOG2PI * bn)
    lpb = jnp.float32(0.0)
    hs = []
    for c in range(nchunk):
      loc = outr[c][...]
      ep = epsr[c][...]
      sp = jnp.maximum(loc, 0.0) + jnp.log1p(jnp.exp(-jnp.abs(loc)))
      scl = sp + 0.01
      z = loc + scl * ep
      lqb += jnp.sum(-0.5 * ep * ep - jnp.log(scl))
      cs = pl.ds(c * 128, 128)
      mx = None
      for kk in range(5):
        t = (z - pm_ref[kk:kk + 1, cs]) / jnp.exp(pls_ref[kk:kk + 1, cs])
        ck = (-0.5 * t * t - pls_ref[kk:kk + 1, cs]
              + (lw_ref[0, kk] - 0.5 * LOG2PI))
        mx = ck if kk == 0 else jnp.maximum(mx, ck)
      se = jnp.zeros_like(mx)
      for kk in range(5):
        t = (z - pm_ref[kk:kk + 1, cs]) / jnp.exp(pls_ref[kk:kk + 1, cs])
        ck = (-0.5 * t * t - pls_ref[kk:kk + 1, cs]
              + (lw_ref[0, kk] - 0.5 * LOG2PI))
        se = se + jnp.exp(ck - mx)
      lpb += jnp.sum(mx + jnp.log(se))
      hs.append(jnp.where(z > 0, z, jnp.expm1(z)))

    if is_t2:
      a2 = jnp.zeros((bn, 2), F32)
      for c2 in range(2):
        acc2 = jnp.zeros((bn, 128), F32)
        for c in range(nchunk):
          acc2 += jnp.dot(hs[c],
                          wn_ref[pl.ds(c * 128, 128), pl.ds(c2 * 128, 128)],
                          preferred_element_type=F32)
        xw2_ref[c2] = acc2
        a2 += jnp.dot(acc2, ap_ref[pl.ds(c2 * 128, 128), :],
                      preferred_element_type=F32)
      ai2_ref[...] = a2[:, 0:1]
      aj2_ref[...] = a2[:, 1:2]
    else:
      lg = jnp.zeros((bn, 40), F32)
      for c2 in range(nchunk):
        lg += lax.dot_general(hs[c2], wn_ref[:, pl.ds(c2 * 128, 128)],
                              (((1,), (1,)), ((), ())),
                              preferred_element_type=F32)
      logit_ref[...] = lg + fcb_ref[...]

    @pl.when(i == 0)
    def _():
      kl_ref[0, 0] = 0.0
      lq_ref[0, 0] = 0.0
      lp_ref[0, 0] = 0.0
      if nr_ref is not None:
        nr_ref[0, 0] = 0.0

    kl_ref[0, 0] += klb
    lq_ref[0, 0] += lqb
    lp_ref[0, 0] += lpb
    if nr_ref is not None:
      nr_ref[0, 0] += jnp.sum(deg)

  sc = jax.ShapeDtypeStruct((1, 1), F32)

  def _out_map(c):
    return lambda i: (c * nblocks + i, 0)

  def _eps_map(c):
    return lambda i: (i, c)

  in_specs = (
      [pl.BlockSpec((bn, 128), _out_map(c)) for c in range(nchunk)]
      + [pl.BlockSpec((bn, 128), _eps_map(c)) for c in range(nchunk)]
      + [pl.BlockSpec((5, d_model), lambda i: (0, 0)),
         pl.BlockSpec((5, d_model), lambda i: (0, 0)),
         pl.BlockSpec((1, 5), lambda i: (0, 0)),
         pl.BlockSpec((bn, 16), lambda i: (i, 0)),
         pl.BlockSpec(w_next.shape, lambda i: (0, 0))]
  )
  if is_t2:
    in_specs.append(pl.BlockSpec((256, 2), lambda i: (0, 0)))
    out_specs = [
        pl.BlockSpec((2, bn, 128), lambda i: (0, i, 0)),
        pl.BlockSpec((bn, 1), lambda i: (i, 0)),
        pl.BlockSpec((bn, 1), lambda i: (i, 0)),
        pl.BlockSpec((1, 1), lambda i: (0, 0)),
        pl.BlockSpec((1, 1), lambda i: (0, 0)),
        pl.BlockSpec((1, 1), lambda i: (0, 0)),
        pl.BlockSpec((1, 1), lambda i: (0, 0)),
    ]
    out_shape = [jax.ShapeDtypeStruct((2, N, 128), F32),
                 jax.ShapeDtypeStruct((N, 1), F32),
                 jax.ShapeDtypeStruct((N, 1), F32), sc, sc, sc, sc]
    extra = (ap_next,)
  else:
    in_specs.append(pl.BlockSpec((1, 40), lambda i: (0, 0)))
    out_specs = [
        pl.BlockSpec((bn, 40), lambda i: (i, 0)),
        pl.BlockSpec((1, 1), lambda i: (0, 0)),
        pl.BlockSpec((1, 1), lambda i: (0, 0)),
        pl.BlockSpec((1, 1), lambda i: (0, 0)),
    ]
    out_shape = [jax.ShapeDtypeStruct((N, 40), F32), sc, sc, sc]
    extra = (fcb,)

  return pl.pallas_call(
      body,
      grid=(nblocks,),
      in_specs=in_specs,
      out_specs=out_specs,
      out_shape=out_shape,
  )(*([out_flat] * nchunk + [eps2d] * nchunk),
    pm, pls, logw, stats, w_next, *extra)


def kernel(x, edge_index, W1, att1, pm1, pls1, plg1,
           W2, att2, pm2, pls2, plg2, fc_w, fc_b):
  src0, dst0 = edge_index[0], edge_index[1]
  loops = jnp.arange(N, dtype=src0.dtype)
  padz = jnp.zeros((EP - ESL,), src0.dtype)
  src = jnp.concatenate([src0, loops, padz])
  dst = jnp.concatenate([dst0, loops, padz])

  atti = att1[0, :, :256]
  attj = att1[0, :, 256:]
  eye = jnp.eye(4, dtype=F32)
  ap_i = (eye[:, :, None] * atti[:, None, :]).reshape(4, 1024).T
  ap_j = (eye[:, :, None] * attj[:, None, :]).reshape(4, 1024).T
  ap1 = jnp.concatenate([ap_i, ap_j], axis=1)
  ap2 = jnp.concatenate([att2[0, 0, :256, None],
                         att2[0, 0, 256:, None]], axis=1)

  xw1, ai1, aj1 = _t1_call(x, W1, ap1)

  statsp1, ex1 = _b1_call(4, ai1.reshape(-1), aj1.reshape(-1), src, dst)
  merged1, den1 = _tm_call(4, statsp1)
  alpha1 = _b2_call(4, den1.reshape(-1), dst, ex1)
  out1 = _c_call(8, 4, xw1, src, dst, alpha1)

  eps1 = jax.random.normal(jax.random.key(1), (1, N, 1024), F32)[0]
  logw1 = jax.nn.log_softmax(plg1).reshape(1, 5)
  xw2c, ai2, aj2, kl1, nr, lq1, lp1 = _node_stage(
      8, 4, True, out1, eps1, pm1, pls1, logw1, merged1, W2, ap2, None)

  statsp2, ex2 = _b1_call(1, ai2.reshape(-1), aj2.reshape(-1), src, dst)
  merged2, den2 = _tm_call(1, statsp2)
  alpha2 = _b2_call(1, den2.reshape(-1), dst, ex2)
  out2 = _c_call(2, 1, xw2c.reshape(2 * N, 128), src, dst, alpha2)

  eps2 = jax.random.normal(jax.random.key(2), (1, N, 256), F32)[0]
  logw2 = jax.nn.log_softmax(plg2).reshape(1, 5)
  logits, kl2, lq2, lp2 = _node_stage(
      2, 1, False, out2, eps2, pm2, pls2, logw2, merged2, fc_w,
      None, fc_b.reshape(1, 40))

  nrows = nr[0, 0]
  skl = kl1[0, 0] / (nrows * 4.0) + kl2[0, 0] / nrows
  ixz = (lq1[0, 0] - lp1[0, 0]) / N + (lq2[0, 0] - lp2[0, 0]) / N
  return (logits, skl, ixz)


# speedup vs baseline: 2.9844x; 2.9844x over previous
import functools
import math

import jax
import jax.numpy as jnp
from jax import lax
from jax.experimental import pallas as pl
from jax.experimental.pallas import tpu as pltpu
from jax.experimental.pallas import tpu_sc as plsc

N = 10000
ERAW = 160000
ESL = ERAW + N
EP = 172032
NC = 2
NS = 16
B = 64
BC = 128
EPT = EP // (NC * NS)
NB = EPT // B
EPT2 = EP // NS
NB2 = EPT2 // BC
NP = 10240
RPT = NP // NS
LOG2PI = math.log(2.0 * math.pi)
F32 = jnp.float32
I32 = jnp.int32

_mesh = functools.partial(
    plsc.VectorSubcoreMesh, core_axis_name="c", subcore_axis_name="s",
    num_cores=NC, num_subcores=NS)
_sc_params = pltpu.CompilerParams(needs_layout_passes=False)


def _b1_call(H, td, ts, src, dst):
  def body(td_hbm, ts_hbm, src_hbm, dst_hbm, statsp_hbm, ex_hbm,
           srcb, dstb, adr, contrib, vbuf, acc_sh):
    cid = lax.axis_index("c")
    sid = lax.axis_index("s")
    wid = cid * NS + sid

    def zr(i, _):
      for f in range(8):
        adr[i, pl.ds(f * 16, 16)] = jnp.zeros((16,), F32)
      return 0

    lax.fori_loop(0, B, zr, 0)
    for z in range(RPT // B):
      pltpu.sync_copy(adr, acc_sh.at[pl.ds(sid * RPT + z * B, B)])
    plsc.subcore_barrier()

    def batch(b, _):
      li = lax.iota(I32, 16)
      one = jnp.full((16,), 1.0, F32)
      zero = jnp.zeros((16,), F32)
      m_ex = li < H
      m_ar = (li >= 4) & (li < 4 + H)
      m_v = li == 8
      base = wid * EPT + b * B
      pltpu.sync_copy(src_hbm.at[pl.ds(base, B)], srcb)
      pltpu.sync_copy(dst_hbm.at[pl.ds(base, B)], dstb)
      pltpu.sync_copy(td_hbm.at[dstb], contrib)
      pltpu.sync_copy(ts_hbm.at[srcb], adr)
      for g in range(B // 16):
        sv = srcb[pl.ds(g * 16, 16)]
        dv = dstb[pl.ds(g * 16, 16)]
        ev = base + g * 16 + li
        validf = jnp.where(ev < ERAW, jnp.where(sv != dv, one, zero),
                           jnp.where(ev < ESL, one, zero))
        vbuf[pl.ds(g * 16, 16)] = validf
      for e in range(B):
        vrow = contrib[e, pl.ds(0, 16)] + adr[e, pl.ds(0, 16)]
        vb = plsc.load_gather(vbuf, [jnp.full((16,), e, I32)])
        ar = jnp.where(vrow >= 0, vrow, 0.2 * vrow)
        ex = jnp.exp(ar)
        row = jnp.where(m_ar, ex * ar, jnp.where(m_ex, ex, zero)) * vb
        row = jnp.where(m_v, vb, row)
        contrib[e, pl.ds(0, 16)] = row
      pltpu.sync_copy(contrib, ex_hbm.at[pl.ds(base, B)])
      pltpu.sync_copy(contrib, acc_sh.at[dstb], add=True)
      return 0

    lax.fori_loop(0, NB, batch, 0)
    plsc.subcore_barrier()
    for z in range(RPT // B):
      pltpu.sync_copy(acc_sh.at[pl.ds(sid * RPT + z * B, B)], adr)
      pltpu.sync_copy(
          adr, statsp_hbm.at[pl.ds(cid * NP + sid * RPT + z * B, B)])

  fn = pl.kernel(
      body,
      out_type=[jax.ShapeDtypeStruct((NC * NP, 128), F32),
                jax.ShapeDtypeStruct((EP, 128), F32)],
      mesh=_mesh(),
      compiler_params=_sc_params,
      scratch_types=[
          pltpu.VMEM((B,), I32), pltpu.VMEM((B,), I32),
          pltpu.VMEM((B, 128), F32), pltpu.VMEM((B, 128), F32),
          pltpu.VMEM((B,), F32),
          pltpu.MemorySpace.VMEM_SHARED((NP, 128), F32),
      ],
  )
  return fn(td, ts, src, dst)


def _tm_call(statsp):
  def body(p0_ref, p1_ref, m_ref, d_ref):
    m = p0_ref[...] + p1_ref[...]
    m_ref[...] = m[:, 0:16]
    d_ref[...] = m

  nb = NP // 1024
  return pl.pallas_call(
      body,
      grid=(nb,),
      in_specs=[pl.BlockSpec((1024, 128), lambda i: (i, 0)),
                pl.BlockSpec((1024, 128), lambda i: (nb + i, 0))],
      out_specs=[pl.BlockSpec((1024, 16), lambda i: (i, 0)),
                 pl.BlockSpec((1024, 128), lambda i: (i, 0))],
      out_shape=[jax.ShapeDtypeStruct((NP, 16), F32),
                 jax.ShapeDtypeStruct((NP, 128), F32)],
  )(statsp, statsp)


def _b2_call(H, den128, dst, ex):
  def body(den_hbm, dst_hbm, ex_hbm, alpha_hbm, dstb, denr, exb, alb):
    cid = lax.axis_index("c")
    sid = lax.axis_index("s")
    wid = cid * NS + sid

    def batch(b, _):
      base = wid * EPT + b * B
      pltpu.sync_copy(dst_hbm.at[pl.ds(base, B)], dstb)
      pltpu.sync_copy(den_hbm.at[dstb], denr)
      pltpu.sync_copy(ex_hbm.at[pl.ds(base, B)], exb)
      for e in range(B):
        dn = denr[e, pl.ds(0, 16)]
        exv = exb[e, pl.ds(0, 16)]
        alb[pl.ds(e * 16, 16)] = exv / (dn + 1e-16)
      pltpu.sync_copy(alb, alpha_hbm.at[pl.ds(base * 16, B * 16)])
      return 0

    lax.fori_loop(0, NB, batch, 0)

  fn = pl.kernel(
      body,
      out_type=[jax.ShapeDtypeStruct((EP * 16,), F32)],
      mesh=_mesh(),
      compiler_params=_sc_params,
      scratch_types=[
          pltpu.VMEM((B,), I32),
          pltpu.VMEM((B, 128), F32),
          pltpu.VMEM((B, 128), F32),
          pltpu.VMEM((B * 16,), F32),
      ],
  )
  return fn(den128, dst, ex)[0]


def _c_call(nchunk, H, xw, src, dst, alpha1d):
  cpsc = nchunk // NC
  cph = nchunk // H

  def body(xw_hbm, src_hbm, dst_hbm, al_hbm, out_hbm,
           srcb, src2, dstb, alb, rows_v, acc_sh):
    cid = lax.axis_index("c")
    sid = lax.axis_index("s")

    def chunk_body(cc, _):
      chunk = cid * cpsc + cc
      hh = chunk // cph

      def zr(i, _):
        for f in range(8):
          rows_v[i, pl.ds(f * 16, 16)] = jnp.zeros((16,), F32)
        return 0

      lax.fori_loop(0, BC, zr, 0)
      for z in range(RPT // BC):
        pltpu.sync_copy(rows_v, acc_sh.at[pl.ds(sid * RPT + z * BC, BC)])
      plsc.subcore_barrier()

      def batch(b, _):
        base = sid * EPT2 + b * BC
        pltpu.sync_copy(src_hbm.at[pl.ds(base, BC)], srcb)
        pltpu.sync_copy(dst_hbm.at[pl.ds(base, BC)], dstb)
        pltpu.sync_copy(al_hbm.at[pl.ds(base * 16, BC * 16)], alb)
        coff = chunk * N
        for g in range(BC // 16):
          src2[pl.ds(g * 16, 16)] = srcb[pl.ds(g * 16, 16)] + coff
        pltpu.sync_copy(xw_hbm.at[src2], rows_v)
        onehot = jnp.where(lax.iota(I32, 16) == hh,
                           jnp.full((16,), 1.0, F32), jnp.zeros((16,), F32))
        for e in range(BC):
          av = jnp.sum(alb[pl.ds(e * 16, 16)] * onehot)
          for f in range(8):
            rows_v[e, pl.ds(f * 16, 16)] = rows_v[e, pl.ds(f * 16, 16)] * av
        pltpu.sync_copy(rows_v, acc_sh.at[dstb], add=True)
        return 0

      lax.fori_loop(0, NB2, batch, 0)
      plsc.subcore_barrier()
      for z in range(RPT // BC):
        pltpu.sync_copy(acc_sh.at[pl.ds(sid * RPT + z * BC, BC)], rows_v)
        pltpu.sync_copy(
            rows_v, out_hbm.at[pl.ds(chunk * NP + sid * RPT + z * BC, BC)])
      plsc.subcore_barrier()
      return 0

    lax.fori_loop(0, cpsc, chunk_body, 0)

  fn = pl.kernel(
      body,
      out_type=[jax.ShapeDtypeStruct((nchunk * NP, 128), F32)],
      mesh=_mesh(),
      compiler_params=_sc_params,
      scratch_types=[
          pltpu.VMEM((BC,), I32), pltpu.VMEM((BC,), I32),
          pltpu.VMEM((BC,), I32), pltpu.VMEM((BC * 16,), F32),
          pltpu.VMEM((BC, 128), F32),
          pltpu.MemorySpace.VMEM_SHARED((NP, 128), F32),
      ],
  )
  return fn(xw, src, dst, alpha1d)[0]


def _t1_call(x, W1, ap1):
  def body(x_ref, w_ref, ap_ref, xw_ref, td_ref, ts_ref):
    c = pl.program_id(1)
    xc = jnp.dot(x_ref[...], w_ref[...], preferred_element_type=F32)
    xw_ref[...] = xc
    part = jnp.dot(xc, ap_ref[...], preferred_element_type=F32)
    ai = part[:, 0:4]
    aj = part[:, 4:8]
    zpad = jnp.zeros((1000, 120), F32)
    td = jnp.concatenate([ai, ai, zpad], axis=1)
    ts = jnp.concatenate([aj, aj, zpad], axis=1)

    @pl.when(c == 0)
    def _():
      td_ref[...] = td
      ts_ref[...] = ts

    @pl.when(c != 0)
    def _():
      td_ref[...] += td
      ts_ref[...] += ts

  return pl.pallas_call(
      body,
      grid=(10, 8),
      in_specs=[
          pl.BlockSpec((1000, 128), lambda i, c: (i, 0)),
          pl.BlockSpec((128, 128), lambda i, c: (0, c)),
          pl.BlockSpec((128, 8), lambda i, c: (c, 0)),
      ],
      out_specs=[
          pl.BlockSpec((1000, 128), lambda i, c: (c * 10 + i, 0)),
          pl.BlockSpec((1000, 128), lambda i, c: (i, 0)),
          pl.BlockSpec((1000, 128), lambda i, c: (i, 0)),
      ],
      out_shape=[jax.ShapeDtypeStruct((8 * N, 128), F32),
                 jax.ShapeDtypeStruct((N, 128), F32),
                 jax.ShapeDtypeStruct((N, 128), F32)],
  )(x, W1, ap1)


def _node_stage(nchunk, H, is_t2, out3d, eps2d, pm, pls, logw,
                stats, w_next, ap_next, fcb):
  bn = 400
  nblocks = N // bn
  d_model = nchunk * 128

  def body(*refs):
    outr = refs[0:nchunk]
    epsr = refs[nchunk:2 * nchunk]
    pm_ref, pls_ref, lw_ref, st_ref, wn_ref = refs[2 * nchunk:2 * nchunk + 5]
    k = 2 * nchunk + 5
    if is_t2:
      ap_ref = refs[k]
      xw2_ref, td2_ref, ts2_ref, kl_ref, nr_ref, lq_ref, lp_ref = refs[k + 1:]
    else:
      fcb_ref = refs[k]
      logit_ref, kl_ref, lq_ref, lp_ref = refs[k + 1:]
      nr_ref = None
    i = pl.program_id(0)

    st = st_ref[...]
    den = st[:, 0:H]
    s1 = st[:, 4:4 + H]
    deg = st[:, 8:9]
    klb = jnp.sum(s1 / den - jnp.log(den)) + H * jnp.sum(jnp.log(deg))

    lqb = jnp.float32(-0.5 * d_model * LOG2PI * bn)
    lpb = jnp.float32(0.0)
    hs = []
    for c in range(nchunk):
      loc = outr[c][0]
      ep = epsr[c][...]
      sp = jnp.maximum(loc, 0.0) + jnp.log1p(jnp.exp(-jnp.abs(loc)))
      scl = sp + 0.01
      z = loc + scl * ep
      lqb += jnp.sum(-0.5 * ep * ep - jnp.log(scl))
      cs = pl.ds(c * 128, 128)
      mx = None
      for kk in range(5):
        t = (z - pm_ref[kk:kk + 1, cs]) / jnp.exp(pls_ref[kk:kk + 1, cs])
        ck = (-0.5 * t * t - pls_ref[kk:kk + 1, cs]
              + (lw_ref[0, kk] - 0.5 * LOG2PI))
        mx = ck if kk == 0 else jnp.maximum(mx, ck)
      se = jnp.zeros_like(mx)
      for kk in range(5):
        t = (z - pm_ref[kk:kk + 1, cs]) / jnp.exp(pls_ref[kk:kk + 1, cs])
        ck = (-0.5 * t * t - pls_ref[kk:kk + 1, cs]
              + (lw_ref[0, kk] - 0.5 * LOG2PI))
        se = se + jnp.exp(ck - mx)
      lpb += jnp.sum(mx + jnp.log(se))
      hs.append(jnp.where(z > 0, z, jnp.exp(jnp.minimum(z, 0.0)) - 1.0))

    if is_t2:
      a2 = jnp.zeros((bn, 2), F32)
      for c2 in range(2):
        acc2 = jnp.zeros((bn, 128), F32)
        for c in range(nchunk):
          acc2 += jnp.dot(hs[c],
                          wn_ref[pl.ds(c * 128, 128), pl.ds(c2 * 128, 128)],
                          preferred_element_type=F32)
        xw2_ref[c2] = acc2
        a2 += jnp.dot(acc2, ap_ref[pl.ds(c2 * 128, 128), :],
                      preferred_element_type=F32)
      zpad3 = jnp.zeros((bn, 3), F32)
      zpad123 = jnp.zeros((bn, 123), F32)
      td2_ref[...] = jnp.concatenate(
          [a2[:, 0:1], zpad3, a2[:, 0:1], zpad123], axis=1)
      ts2_ref[...] = jnp.concatenate(
          [a2[:, 1:2], zpad3, a2[:, 1:2], zpad123], axis=1)
    else:
      lg = jnp.zeros((bn, 40), F32)
      for c2 in range(nchunk):
        lg += lax.dot_general(hs[c2], wn_ref[:, pl.ds(c2 * 128, 128)],
                              (((1,), (1,)), ((), ())),
                              preferred_element_type=F32)
      logit_ref[...] = lg + fcb_ref[...]

    z11 = jnp.zeros((1, 1), F32)

    @pl.when(i == 0)
    def _():
      kl_ref[...] = z11
      lq_ref[...] = z11
      lp_ref[...] = z11
      if nr_ref is not None:
        nr_ref[...] = z11

    kl_ref[...] += klb[None, None]
    lq_ref[...] += lqb[None, None]
    lp_ref[...] += lpb[None, None]
    if nr_ref is not None:
      nr_ref[...] += jnp.sum(deg)[None, None]

  sc = jax.ShapeDtypeStruct((1, 1), F32)

  def _out_map(c):
    return lambda i: (c, i, 0)

  def _eps_map(c):
    return lambda i: (i, c)

  in_specs = (
      [pl.BlockSpec((1, bn, 128), _out_map(c)) for c in range(nchunk)]
      + [pl.BlockSpec((bn, 128), _eps_map(c)) for c in range(nchunk)]
      + [pl.BlockSpec((5, d_model), lambda i: (0, 0)),
         pl.BlockSpec((5, d_model), lambda i: (0, 0)),
         pl.BlockSpec((1, 5), lambda i: (0, 0)),
         pl.BlockSpec((bn, 16), lambda i: (i, 0)),
         pl.BlockSpec(w_next.shape, lambda i: (0, 0))]
  )
  if is_t2:
    in_specs.append(pl.BlockSpec((256, 2), lambda i: (0, 0)))
    out_specs = [
        pl.BlockSpec((2, bn, 128), lambda i: (0, i, 0)),
        pl.BlockSpec((bn, 128), lambda i: (i, 0)),
        pl.BlockSpec((bn, 128), lambda i: (i, 0)),
        pl.BlockSpec((1, 1), lambda i: (0, 0)),
        pl.BlockSpec((1, 1), lambda i: (0, 0)),
        pl.BlockSpec((1, 1), lambda i: (0, 0)),
        pl.BlockSpec((1, 1), lambda i: (0, 0)),
    ]
    out_shape = [jax.ShapeDtypeStruct((2, N, 128), F32),
                 jax.ShapeDtypeStruct((N, 128), F32),
                 jax.ShapeDtypeStruct((N, 128), F32), sc, sc, sc, sc]
    extra = (ap_next,)
  else:
    in_specs.append(pl.BlockSpec((1, 40), lambda i: (0, 0)))
    out_specs = [
        pl.BlockSpec((bn, 40), lambda i: (i, 0)),
        pl.BlockSpec((1, 1), lambda i: (0, 0)),
        pl.BlockSpec((1, 1), lambda i: (0, 0)),
        pl.BlockSpec((1, 1), lambda i: (0, 0)),
    ]
    out_shape = [jax.ShapeDtypeStruct((N, 40), F32), sc, sc, sc]
    extra = (fcb,)

  return pl.pallas_call(
      body,
      grid=(nblocks,),
      in_specs=in_specs,
      out_specs=out_specs,
      out_shape=out_shape,
  )(*([out3d] * nchunk + [eps2d] * nchunk),
    pm, pls, logw, stats, w_next, *extra)


def kernel(x, edge_index, W1, att1, pm1, pls1, plg1,
           W2, att2, pm2, pls2, plg2, fc_w, fc_b):
  src0, dst0 = edge_index[0], edge_index[1]
  loops = jnp.arange(N, dtype=src0.dtype)
  padz = jnp.zeros((EP - ESL,), src0.dtype)
  src = jnp.concatenate([src0, loops, padz])
  dst = jnp.concatenate([dst0, loops, padz])

  atti = att1[0, :, :256]
  attj = att1[0, :, 256:]
  eye = jnp.eye(4, dtype=F32)
  ap_i = (eye[:, :, None] * atti[:, None, :]).reshape(4, 1024).T
  ap_j = (eye[:, :, None] * attj[:, None, :]).reshape(4, 1024).T
  ap1 = jnp.concatenate([ap_i, ap_j], axis=1)
  ap2 = jnp.concatenate([att2[0, 0, :256, None],
                         att2[0, 0, 256:, None]], axis=1)

  xw1, tdi1, tsj1 = _t1_call(x, W1, ap1)

  statsp1, ex1 = _b1_call(4, tdi1, tsj1, src, dst)
  merged1, den128_1 = _tm_call(statsp1)
  alpha1 = _b2_call(4, den128_1, dst, ex1)
  out1 = _c_call(8, 4, xw1, src, dst, alpha1)

  eps1 = jax.random.normal(jax.random.key(1), (1, N, 1024), F32)[0]
  logw1 = jax.nn.log_softmax(plg1).reshape(1, 5)
  xw2c, tdi2, tsj2, kl1, nr, lq1, lp1 = _node_stage(
      8, 4, True, out1.reshape(8, NP, 128), eps1, pm1, pls1, logw1,
      merged1, W2, ap2, None)

  statsp2, ex2 = _b1_call(1, tdi2, tsj2, src, dst)
  merged2, den128_2 = _tm_call(statsp2)
  alpha2 = _b2_call(1, den128_2, dst, ex2)
  out2 = _c_call(2, 1, xw2c.reshape(2 * N, 128), src, dst, alpha2)

  eps2 = jax.random.normal(jax.random.key(2), (1, N, 256), F32)[0]
  logw2 = jax.nn.log_softmax(plg2).reshape(1, 5)
  logits, kl2, lq2, lp2 = _node_stage(
      2, 1, False, out2.reshape(2, NP, 128), eps2, pm2, pls2, logw2,
      merged2, fc_w, None, fc_b.reshape(1, 40))

  nrows = nr[0, 0]
  skl = kl1[0, 0] / (nrows * 4.0) + kl2[0, 0] / nrows
  ixz = (lq1[0, 0] - lp1[0, 0]) / N + (lq2[0, 0] - lp2[0, 0]) / N
  return (logits, skl, ixz)
